# baseline (speedup 1.0000x reference)
import functools

import jax
import jax.numpy as jnp
import numpy as np
from jax import lax
from jax.experimental import pallas as pl
from jax.experimental.pallas import tpu as pltpu
from jax.experimental.pallas import tpu_sc as plsc

N = 10000
E = 320000
D = 128
G = 128
OUT = 10

NC = 2
NS = 16
NW = NC * NS

N_PAD = 10240
AGG_ROWS = N_PAD + 8
CH = 128
EPW = 10240
E_PAD = EPW * NW
NSTEPS = EPW // CH
NH = 2
IH = NSTEPS // NH
ZPS = N_PAD // NS

GP = G + 8
RPW = N_PAD // NW
PCH = 64
PSTEPS = RPW // PCH

_BN_INV = float(1.0 / np.sqrt(1.0 + 1e-5))

_mesh = plsc.VectorSubcoreMesh(core_axis_name="c", subcore_axis_name="s")


def _zero_rows(ref, nrows):
    z16 = jnp.zeros((16,), jnp.float32)

    @pl.loop(0, nrows)
    def _(r):
        @pl.loop(0, D // 16)
        def _(j):
            ref[r, pl.ds(j * 16, 16)] = z16


@functools.partial(
    pl.kernel,
    out_type=jax.ShapeDtypeStruct((NC * N_PAD, D), jnp.float32),
    mesh=_mesh,
    scratch_types=[
        pltpu.VMEM((IH, CH), jnp.int32),
        pltpu.VMEM((IH, CH), jnp.int32),
        pltpu.VMEM((CH, D), jnp.float32),
        pltpu.VMEM((CH, D), jnp.float32),
        pltpu.VMEM_SHARED((AGG_ROWS, D), jnp.float32),
        pltpu.SemaphoreType.DMA,
        pltpu.SemaphoreType.DMA,
    ],
)
def _sc_segsum(h_hbm, src_hbm, dst_hbm, out_hbm, src_v, dst_v, rows0_v, rows1_v,
               agg_sh, sem0, sem1):
    c = lax.axis_index("c")
    s = lax.axis_index("s")
    wid = c * NS + s

    _zero_rows(rows0_v, CH)

    @pl.loop(0, ZPS // CH)
    def _(i):
        pltpu.sync_copy(rows0_v, agg_sh.at[pl.ds(s * ZPS + i * CH, CH)])

    plsc.subcore_barrier()

    @pl.loop(0, NH)
    def _(hh):
        pltpu.sync_copy(src_hbm.at[pl.ds(wid * NSTEPS + hh * IH, IH)], src_v)
        pltpu.sync_copy(dst_hbm.at[pl.ds(wid * NSTEPS + hh * IH, IH)], dst_v)

        pltpu.async_copy(h_hbm.at[src_v.at[0]], rows0_v, sem0)

        @pl.loop(0, IH // 2)
        def _(i):
            t0 = i * 2
            pltpu.async_copy(h_hbm.at[src_v.at[t0 + 1]], rows1_v, sem1)
            pltpu.make_async_copy(h_hbm.at[src_v.at[t0]], rows0_v, sem0).wait()
            pltpu.sync_copy(rows0_v, agg_sh.at[dst_v.at[t0]], add=True)

            @pl.when(i + 1 < IH // 2)
            def _():
                pltpu.async_copy(h_hbm.at[src_v.at[t0 + 2]], rows0_v, sem0)

            pltpu.make_async_copy(h_hbm.at[src_v.at[t0 + 1]], rows1_v, sem1).wait()
            pltpu.sync_copy(rows1_v, agg_sh.at[dst_v.at[t0 + 1]], add=True)

    plsc.subcore_barrier()
    pltpu.sync_copy(
        agg_sh.at[pl.ds(s * ZPS, ZPS)],
        out_hbm.at[pl.ds(c * N_PAD + s * ZPS, ZPS)],
    )


@functools.partial(
    pl.kernel,
    out_type=jax.ShapeDtypeStruct((NC * G, D), jnp.float32),
    mesh=_mesh,
    scratch_types=[
        pltpu.VMEM((PCH,), jnp.int32),
        pltpu.VMEM((PCH, D), jnp.float32),
        pltpu.VMEM_SHARED((GP, D), jnp.float32),
    ],
)
def _sc_pool(h_hbm, batch_hbm, out_hbm, bidx_v, rows_v, pool_sh):
    c = lax.axis_index("c")
    s = lax.axis_index("s")
    wid = c * NS + s

    _zero_rows(rows_v, 8)
    pltpu.sync_copy(rows_v.at[pl.ds(0, 8)], pool_sh.at[pl.ds(s * 8, 8)])
    plsc.subcore_barrier()

    rbase = wid * RPW

    @pl.loop(0, PSTEPS)
    def _(t):
        off = pl.multiple_of(rbase + t * PCH, PCH)
        pltpu.sync_copy(h_hbm.at[pl.ds(off, PCH)], rows_v)
        pltpu.sync_copy(batch_hbm.at[pl.ds(off, PCH)], bidx_v)
        pltpu.sync_copy(rows_v, pool_sh.at[bidx_v], add=True)

    plsc.subcore_barrier()
    pltpu.sync_copy(
        pool_sh.at[pl.ds(s * 8, 8)],
        out_hbm.at[pl.ds(c * G + s * 8, 8)],
    )


BLK = 1024
NB = N_PAD // BLK


def _dense_body(h_ref, p0_ref, p1_ref, w1_ref, b1_ref, g_ref, be_ref, w2_ref,
                b2_ref, o_ref):
    z = h_ref[...] + p0_ref[...] + p1_ref[...]
    z = jnp.dot(z, w1_ref[...], preferred_element_type=jnp.float32) + b1_ref[...]
    z = z * (g_ref[...] * _BN_INV) + be_ref[...]
    z = jnp.maximum(z, 0.0)
    z = jnp.dot(z, w2_ref[...], preferred_element_type=jnp.float32) + b2_ref[...]
    o_ref[...] = jnp.maximum(z, 0.0)


def _tc_dense(h, parts, W1, b1, gamma, beta, W2, b2):
    vspec = pl.BlockSpec((1, D), lambda i: (0, 0))
    wspec = pl.BlockSpec((D, D), lambda i: (0, 0))
    return pl.pallas_call(
        _dense_body,
        grid=(NB,),
        in_specs=[
            pl.BlockSpec((BLK, D), lambda i: (i, 0)),
            pl.BlockSpec((BLK, D), lambda i: (i, 0)),
            pl.BlockSpec((BLK, D), lambda i: (i + NB, 0)),
            wspec, vspec, vspec, vspec, wspec, vspec,
        ],
        out_specs=pl.BlockSpec((BLK, D), lambda i: (i, 0)),
        out_shape=jax.ShapeDtypeStruct((N_PAD, D), jnp.float32),
    )(h, parts, parts, W1, b1.reshape(1, D), gamma.reshape(1, D),
      beta.reshape(1, D), W2, b2.reshape(1, D))


def _head_body(p_ref, w1_ref, b1_ref, w2_ref, b2_ref, o_ref):
    z = p_ref[pl.ds(0, G), :] + p_ref[pl.ds(G, G), :]
    z = jnp.dot(z, w1_ref[...], preferred_element_type=jnp.float32) + b1_ref[...]
    z = jnp.maximum(z, 0.0)
    o_ref[...] = jnp.dot(z, w2_ref[...], preferred_element_type=jnp.float32) + b2_ref[...]


def _tc_head(pparts, W1, b1, W2, b2):
    return pl.pallas_call(
        _head_body,
        out_shape=jax.ShapeDtypeStruct((G, OUT), jnp.float32),
    )(pparts, W1, b1.reshape(1, D), W2, b2.reshape(1, OUT))


def kernel(x, edge_index, batch, conv0_W1, conv0_b1, conv0_gamma, conv0_beta,
           conv0_W2, conv0_b2, conv1_W1, conv1_b1, conv1_gamma, conv1_beta,
           conv1_W2, conv1_b2, conv2_W1, conv2_b1, conv2_gamma, conv2_beta,
           conv2_W2, conv2_b2, mlp_W1, mlp_b1, mlp_W2, mlp_b2):
    src = edge_index[0].astype(jnp.int32)
    dst = edge_index[1].astype(jnp.int32)
    srcp = jnp.pad(src, (0, E_PAD - E)).reshape(NW * NSTEPS, CH)
    dstp = jnp.pad(dst, (0, E_PAD - E),
                   constant_values=N_PAD).reshape(NW * NSTEPS, CH)
    batchp = jnp.pad(batch.astype(jnp.int32), (0, N_PAD - N), constant_values=G)
    h = jnp.pad(x, ((0, N_PAD - N), (0, 0)))

    layers = (
        (conv0_W1, conv0_b1, conv0_gamma, conv0_beta, conv0_W2, conv0_b2),
        (conv1_W1, conv1_b1, conv1_gamma, conv1_beta, conv1_W2, conv1_b2),
        (conv2_W1, conv2_b1, conv2_gamma, conv2_beta, conv2_W2, conv2_b2),
    )
    for (W1, b1, gamma, beta, W2, b2) in layers:
        parts = _sc_segsum(h, srcp, dstp)
        h = _tc_dense(h, parts, W1, b1, gamma, beta, W2, b2)

    pparts = _sc_pool(h, batchp)
    return _tc_head(pparts, mlp_W1, mlp_b1, mlp_W2, mlp_b2)


# speedup vs baseline: 1.1611x; 1.1611x over previous
import functools

import jax
import jax.numpy as jnp
import numpy as np
from jax import lax
from jax.experimental import pallas as pl
from jax.experimental.pallas import tpu as pltpu
from jax.experimental.pallas import tpu_sc as plsc

N = 10000
E = 320000
D = 128
G = 128
OUT = 10

NC = 2
NS = 16
NW = NC * NS

N_PAD = 10240
AGG_ROWS = N_PAD + 8
CH = 128
S0 = 60
S1 = 97
BASE0 = NS * S0 * CH
E_PAD = NS * (S0 + S1) * CH
ZPS = N_PAD // NS

GP = G + 8
RPW = N_PAD // NW
PCH = 64
PSTEPS = RPW // PCH

_BN_INV = float(1.0 / np.sqrt(1.0 + 1e-5))

_mesh = plsc.VectorSubcoreMesh(core_axis_name="c", subcore_axis_name="s")


def _zero_rows(ref, nrows):
    z16 = jnp.zeros((16,), jnp.float32)
    w = ref.shape[1]

    @pl.loop(0, nrows)
    def _(r):
        @pl.loop(0, w // 16)
        def _(j):
            ref[r, pl.ds(j * 16, 16)] = z16


@functools.partial(
    pl.kernel,
    out_type=jax.ShapeDtypeStruct((NC * N_PAD, D), jnp.float32),
    mesh=_mesh,
    scratch_types=[
        pltpu.VMEM((CH,), jnp.int32),
        pltpu.VMEM((CH,), jnp.int32),
        pltpu.VMEM((CH, D), jnp.float32),
        pltpu.VMEM_SHARED((AGG_ROWS, D), jnp.float32),
        pltpu.SemaphoreType.DMA,
    ],
)
def _sc_segsum(h_hbm, src_hbm, dst_hbm, out_hbm, src_v, dst_v, rows_v, agg_sh,
               sem):
    c = lax.axis_index("c")
    s = lax.axis_index("s")

    _zero_rows(rows_v, CH)

    @pl.loop(0, ZPS // CH)
    def _(i):
        pltpu.sync_copy(rows_v, agg_sh.at[pl.ds(s * ZPS + i * CH, CH)])

    plsc.subcore_barrier()

    nst = jnp.where(c == 0, S0, S1)
    base = jnp.where(c == 0, s * (S0 * CH), BASE0 + s * (S1 * CH))

    @pl.loop(0, S1)
    def _(t):
        @pl.when(t < nst)
        def _():
            off = base + t * CH
            pltpu.sync_copy(src_hbm.at[pl.ds(off, CH)], src_v)
            pltpu.sync_copy(dst_hbm.at[pl.ds(off, CH)], dst_v)
            pltpu.async_copy(h_hbm.at[src_v], rows_v, sem).wait()
            pltpu.sync_copy(rows_v, agg_sh.at[dst_v], add=True)

    plsc.subcore_barrier()
    pltpu.sync_copy(
        agg_sh.at[pl.ds(s * ZPS, ZPS)],
        out_hbm.at[pl.ds(c * N_PAD + s * ZPS, ZPS)],
    )


@functools.partial(
    pl.kernel,
    out_type=jax.ShapeDtypeStruct((NC * G, D), jnp.float32),
    mesh=_mesh,
    scratch_types=[
        pltpu.VMEM((PCH,), jnp.int32),
        pltpu.VMEM((PCH, D), jnp.float32),
        pltpu.VMEM_SHARED((GP, D), jnp.float32),
    ],
)
def _sc_pool(h_hbm, batch_hbm, out_hbm, bidx_v, rows_v, pool_sh):
    c = lax.axis_index("c")
    s = lax.axis_index("s")
    wid = c * NS + s

    _zero_rows(rows_v, 8)
    pltpu.sync_copy(rows_v.at[pl.ds(0, 8)], pool_sh.at[pl.ds(s * 8, 8)])
    plsc.subcore_barrier()

    rbase = wid * RPW

    @pl.loop(0, PSTEPS)
    def _(t):
        off = pl.multiple_of(rbase + t * PCH, PCH)
        pltpu.sync_copy(h_hbm.at[pl.ds(off, PCH)], rows_v)
        pltpu.sync_copy(batch_hbm.at[pl.ds(off, PCH)], bidx_v)
        pltpu.sync_copy(rows_v, pool_sh.at[bidx_v], add=True)

    plsc.subcore_barrier()
    pltpu.sync_copy(
        pool_sh.at[pl.ds(s * 8, 8)],
        out_hbm.at[pl.ds(c * G + s * 8, 8)],
    )


BLK = 1024
NB = N_PAD // BLK


def _dense_body(h_ref, p0_ref, p1_ref, w1_ref, b1_ref, g_ref, be_ref, w2_ref,
                b2_ref, o_ref):
    z = h_ref[...] + p0_ref[...] + p1_ref[...]
    z = jnp.dot(z, w1_ref[...], preferred_element_type=jnp.float32) + b1_ref[...]
    z = z * (g_ref[...] * _BN_INV) + be_ref[...]
    z = jnp.maximum(z, 0.0)
    z = jnp.dot(z, w2_ref[...], preferred_element_type=jnp.float32) + b2_ref[...]
    o_ref[...] = jnp.maximum(z, 0.0)


def _tc_dense(h, parts, W1, b1, gamma, beta, W2, b2):
    vspec = pl.BlockSpec((1, D), lambda i: (0, 0))
    wspec = pl.BlockSpec((D, D), lambda i: (0, 0))
    return pl.pallas_call(
        _dense_body,
        grid=(NB,),
        in_specs=[
            pl.BlockSpec((BLK, D), lambda i: (i, 0)),
            pl.BlockSpec((BLK, D), lambda i: (i, 0)),
            pl.BlockSpec((BLK, D), lambda i: (i + NB, 0)),
            wspec, vspec, vspec, vspec, wspec, vspec,
        ],
        out_specs=pl.BlockSpec((BLK, D), lambda i: (i, 0)),
        out_shape=jax.ShapeDtypeStruct((N_PAD, D), jnp.float32),
    )(h, parts, parts, W1, b1.reshape(1, D), gamma.reshape(1, D),
      beta.reshape(1, D), W2, b2.reshape(1, D))


def _head_body(p_ref, w1_ref, b1_ref, w2_ref, b2_ref, o_ref):
    z = p_ref[pl.ds(0, G), :] + p_ref[pl.ds(G, G), :]
    z = jnp.dot(z, w1_ref[...], preferred_element_type=jnp.float32) + b1_ref[...]
    z = jnp.maximum(z, 0.0)
    o_ref[...] = jnp.dot(z, w2_ref[...], preferred_element_type=jnp.float32) + b2_ref[...]


def _tc_head(pparts, W1, b1, W2, b2):
    return pl.pallas_call(
        _head_body,
        out_shape=jax.ShapeDtypeStruct((G, OUT), jnp.float32),
    )(pparts, W1, b1.reshape(1, D), W2, b2.reshape(1, OUT))


def kernel(x, edge_index, batch, conv0_W1, conv0_b1, conv0_gamma, conv0_beta,
           conv0_W2, conv0_b2, conv1_W1, conv1_b1, conv1_gamma, conv1_beta,
           conv1_W2, conv1_b2, conv2_W1, conv2_b1, conv2_gamma, conv2_beta,
           conv2_W2, conv2_b2, mlp_W1, mlp_b1, mlp_W2, mlp_b2):
    src = edge_index[0].astype(jnp.int32)
    dst = edge_index[1].astype(jnp.int32)
    srcp = jnp.pad(src, (0, E_PAD - E))
    dstp = jnp.pad(dst, (0, E_PAD - E), constant_values=N_PAD)
    batchp = jnp.pad(batch.astype(jnp.int32), (0, N_PAD - N), constant_values=G)
    h = jnp.pad(x, ((0, N_PAD - N), (0, 0)))

    layers = (
        (conv0_W1, conv0_b1, conv0_gamma, conv0_beta, conv0_W2, conv0_b2),
        (conv1_W1, conv1_b1, conv1_gamma, conv1_beta, conv1_W2, conv1_b2),
        (conv2_W1, conv2_b1, conv2_gamma, conv2_beta, conv2_W2, conv2_b2),
    )
    for (W1, b1, gamma, beta, W2, b2) in layers:
        parts = _sc_segsum(h, srcp, dstp)
        h = _tc_dense(h, parts, W1, b1, gamma, beta, W2, b2)

    pparts = _sc_pool(h, batchp)
    return _tc_head(pparts, mlp_W1, mlp_b1, mlp_W2, mlp_b2)


# speedup vs baseline: 1.3951x; 1.2015x over previous
import functools

import jax
import jax.numpy as jnp
import numpy as np
from jax import lax
from jax.experimental import pallas as pl
from jax.experimental.pallas import tpu as pltpu
from jax.experimental.pallas import tpu_sc as plsc

N = 10000
E = 320000
D = 128
G = 128
OUT = 10

NC = 2
NS = 16
NW = NC * NS

N_PAD = 10240
AGG_ROWS = N_PAD + 8
CH = 128
S0 = 84
S1 = 73
BASE0 = NS * S0 * CH
E_PAD = NS * (S0 + S1) * CH
ZPS = N_PAD // NS

GP = G + 8
RPW = N_PAD // NW
PCH = 64
PSTEPS = RPW // PCH

_BN_INV = float(1.0 / np.sqrt(1.0 + 1e-5))

_mesh = plsc.VectorSubcoreMesh(core_axis_name="c", subcore_axis_name="s")


def _zero_rows(ref, nrows):
    z16 = jnp.zeros((16,), jnp.float32)
    w = ref.shape[1]

    @pl.loop(0, nrows)
    def _(r):
        @pl.loop(0, w // 16)
        def _(j):
            ref[r, pl.ds(j * 16, 16)] = z16


@functools.partial(
    pl.kernel,
    out_type=jax.ShapeDtypeStruct((NC * N_PAD, D), jnp.float32),
    mesh=_mesh,
    scratch_types=[
        pltpu.VMEM((CH,), jnp.int32),
        pltpu.VMEM((CH,), jnp.int32),
        pltpu.VMEM((CH, D), jnp.float32),
        pltpu.VMEM_SHARED((AGG_ROWS, D), jnp.float32),
        pltpu.SemaphoreType.DMA,
    ],
)
def _sc_segsum(h_hbm, src_hbm, dst_hbm, out_hbm, src_v, dst_v, rows_v, agg_sh,
               sem):
    c = lax.axis_index("c")
    s = lax.axis_index("s")

    _zero_rows(rows_v, CH)

    @pl.loop(0, ZPS // CH)
    def _(i):
        pltpu.sync_copy(rows_v, agg_sh.at[pl.ds(s * ZPS + i * CH, CH)])

    plsc.subcore_barrier()

    nst = jnp.where(c == 0, S0, S1)
    base = jnp.where(c == 0, s * (S0 * CH), BASE0 + s * (S1 * CH))

    @pl.loop(0, S1)
    def _(t):
        @pl.when(t < nst)
        def _():
            off = base + t * CH
            pltpu.sync_copy(src_hbm.at[pl.ds(off, CH)], src_v)
            pltpu.sync_copy(dst_hbm.at[pl.ds(off, CH)], dst_v)
            pltpu.async_copy(h_hbm.at[src_v], rows_v, sem).wait()
            pltpu.sync_copy(rows_v, agg_sh.at[dst_v], add=True)

    plsc.subcore_barrier()
    pltpu.sync_copy(
        agg_sh.at[pl.ds(s * ZPS, ZPS)],
        out_hbm.at[pl.ds(c * N_PAD + s * ZPS, ZPS)],
    )


@functools.partial(
    pl.kernel,
    out_type=jax.ShapeDtypeStruct((NC * G, D), jnp.float32),
    mesh=_mesh,
    scratch_types=[
        pltpu.VMEM((PCH,), jnp.int32),
        pltpu.VMEM((PCH, D), jnp.float32),
        pltpu.VMEM_SHARED((GP, D), jnp.float32),
    ],
)
def _sc_pool(h_hbm, batch_hbm, out_hbm, bidx_v, rows_v, pool_sh):
    c = lax.axis_index("c")
    s = lax.axis_index("s")
    wid = c * NS + s

    _zero_rows(rows_v, 8)
    pltpu.sync_copy(rows_v.at[pl.ds(0, 8)], pool_sh.at[pl.ds(s * 8, 8)])
    plsc.subcore_barrier()

    rbase = wid * RPW

    @pl.loop(0, PSTEPS)
    def _(t):
        off = pl.multiple_of(rbase + t * PCH, PCH)
        pltpu.sync_copy(h_hbm.at[pl.ds(off, PCH)], rows_v)
        pltpu.sync_copy(batch_hbm.at[pl.ds(off, PCH)], bidx_v)
        pltpu.sync_copy(rows_v, pool_sh.at[bidx_v], add=True)

    plsc.subcore_barrier()
    pltpu.sync_copy(
        pool_sh.at[pl.ds(s * 8, 8)],
        out_hbm.at[pl.ds(c * G + s * 8, 8)],
    )


BLK = 1024
NB = N_PAD // BLK


def _dense_body(h_ref, p0_ref, p1_ref, w1_ref, b1_ref, g_ref, be_ref, w2_ref,
                b2_ref, o_ref):
    z = h_ref[...] + p0_ref[...] + p1_ref[...]
    z = jnp.dot(z, w1_ref[...], preferred_element_type=jnp.float32) + b1_ref[...]
    z = z * (g_ref[...] * _BN_INV) + be_ref[...]
    z = jnp.maximum(z, 0.0)
    z = jnp.dot(z, w2_ref[...], preferred_element_type=jnp.float32) + b2_ref[...]
    o_ref[...] = jnp.maximum(z, 0.0)


def _tc_dense(h, parts, W1, b1, gamma, beta, W2, b2):
    vspec = pl.BlockSpec((1, D), lambda i: (0, 0))
    wspec = pl.BlockSpec((D, D), lambda i: (0, 0))
    return pl.pallas_call(
        _dense_body,
        grid=(NB,),
        in_specs=[
            pl.BlockSpec((BLK, D), lambda i: (i, 0)),
            pl.BlockSpec((BLK, D), lambda i: (i, 0)),
            pl.BlockSpec((BLK, D), lambda i: (i + NB, 0)),
            wspec, vspec, vspec, vspec, wspec, vspec,
        ],
        out_specs=pl.BlockSpec((BLK, D), lambda i: (i, 0)),
        out_shape=jax.ShapeDtypeStruct((N_PAD, D), jnp.float32),
    )(h, parts, parts, W1, b1.reshape(1, D), gamma.reshape(1, D),
      beta.reshape(1, D), W2, b2.reshape(1, D))


def _head_body(p_ref, w1_ref, b1_ref, w2_ref, b2_ref, o_ref):
    z = p_ref[pl.ds(0, G), :] + p_ref[pl.ds(G, G), :]
    z = jnp.dot(z, w1_ref[...], preferred_element_type=jnp.float32) + b1_ref[...]
    z = jnp.maximum(z, 0.0)
    o_ref[...] = jnp.dot(z, w2_ref[...], preferred_element_type=jnp.float32) + b2_ref[...]


def _tc_head(pparts, W1, b1, W2, b2):
    return pl.pallas_call(
        _head_body,
        out_shape=jax.ShapeDtypeStruct((G, OUT), jnp.float32),
    )(pparts, W1, b1.reshape(1, D), W2, b2.reshape(1, OUT))


def kernel(x, edge_index, batch, conv0_W1, conv0_b1, conv0_gamma, conv0_beta,
           conv0_W2, conv0_b2, conv1_W1, conv1_b1, conv1_gamma, conv1_beta,
           conv1_W2, conv1_b2, conv2_W1, conv2_b1, conv2_gamma, conv2_beta,
           conv2_W2, conv2_b2, mlp_W1, mlp_b1, mlp_W2, mlp_b2):
    src = edge_index[0].astype(jnp.int32)
    dst = edge_index[1].astype(jnp.int32)
    srcp = jnp.pad(src, (0, E_PAD - E))
    dstp = jnp.pad(dst, (0, E_PAD - E), constant_values=N_PAD)
    batchp = jnp.pad(batch.astype(jnp.int32), (0, N_PAD - N), constant_values=G)
    h = jnp.pad(x, ((0, N_PAD - N), (0, 0)))

    layers = (
        (conv0_W1, conv0_b1, conv0_gamma, conv0_beta, conv0_W2, conv0_b2),
        (conv1_W1, conv1_b1, conv1_gamma, conv1_beta, conv1_W2, conv1_b2),
        (conv2_W1, conv2_b1, conv2_gamma, conv2_beta, conv2_W2, conv2_b2),
    )
    for (W1, b1, gamma, beta, W2, b2) in layers:
        parts = _sc_segsum(h, srcp, dstp)
        h = _tc_dense(h, parts, W1, b1, gamma, beta, W2, b2)

    pparts = _sc_pool(h, batchp)
    return _tc_head(pparts, mlp_W1, mlp_b1, mlp_W2, mlp_b2)


# speedup vs baseline: 1.5106x; 1.0828x over previous
import functools

import jax
import jax.numpy as jnp
import numpy as np
from jax import lax
from jax.experimental import pallas as pl
from jax.experimental.pallas import tpu as pltpu
from jax.experimental.pallas import tpu_sc as plsc

N = 10000
E = 320000
D = 128
G = 128
OUT = 10

NC = 2
NS = 16
NW = NC * NS

N_PAD = 10240
AGG_ROWS = N_PAD + 8
CH = 128
S0 = 93
S1 = 64
BASE0 = NS * S0 * CH
E_PAD = NS * (S0 + S1) * CH
ZPS = N_PAD // NS

GP = G + 8
RPW = N_PAD // NW
PCH = 64
PSTEPS = RPW // PCH

_BN_INV = float(1.0 / np.sqrt(1.0 + 1e-5))

_mesh = plsc.VectorSubcoreMesh(core_axis_name="c", subcore_axis_name="s")


def _zero_rows(ref, nrows):
    z16 = jnp.zeros((16,), jnp.float32)
    w = ref.shape[1]

    @pl.loop(0, nrows)
    def _(r):
        @pl.loop(0, w // 16)
        def _(j):
            ref[r, pl.ds(j * 16, 16)] = z16


@functools.partial(
    pl.kernel,
    out_type=jax.ShapeDtypeStruct((NC * N_PAD, D), jnp.float32),
    mesh=_mesh,
    scratch_types=[
        pltpu.VMEM((CH,), jnp.int32),
        pltpu.VMEM((CH,), jnp.int32),
        pltpu.VMEM((CH, D), jnp.float32),
        pltpu.VMEM_SHARED((AGG_ROWS, D), jnp.float32),
        pltpu.SemaphoreType.DMA,
    ],
)
def _sc_segsum(h_hbm, src_hbm, dst_hbm, out_hbm, src_v, dst_v, rows_v, agg_sh,
               sem):
    c = lax.axis_index("c")
    s = lax.axis_index("s")

    _zero_rows(rows_v, CH)

    @pl.loop(0, ZPS // CH)
    def _(i):
        pltpu.sync_copy(rows_v, agg_sh.at[pl.ds(s * ZPS + i * CH, CH)])

    plsc.subcore_barrier()

    nst = jnp.where(c == 0, S0, S1)
    base = jnp.where(c == 0, s * (S0 * CH), BASE0 + s * (S1 * CH))

    @pl.loop(0, S1)
    def _(t):
        @pl.when(t < nst)
        def _():
            off = base + t * CH
            pltpu.sync_copy(src_hbm.at[pl.ds(off, CH)], src_v)
            pltpu.sync_copy(dst_hbm.at[pl.ds(off, CH)], dst_v)
            pltpu.async_copy(h_hbm.at[src_v], rows_v, sem).wait()
            pltpu.sync_copy(rows_v, agg_sh.at[dst_v], add=True)

    plsc.subcore_barrier()
    pltpu.sync_copy(
        agg_sh.at[pl.ds(s * ZPS, ZPS)],
        out_hbm.at[pl.ds(c * N_PAD + s * ZPS, ZPS)],
    )


@functools.partial(
    pl.kernel,
    out_type=jax.ShapeDtypeStruct((NC * G, D), jnp.float32),
    mesh=_mesh,
    scratch_types=[
        pltpu.VMEM((PCH,), jnp.int32),
        pltpu.VMEM((PCH, D), jnp.float32),
        pltpu.VMEM_SHARED((GP, D), jnp.float32),
    ],
)
def _sc_pool(h_hbm, batch_hbm, out_hbm, bidx_v, rows_v, pool_sh):
    c = lax.axis_index("c")
    s = lax.axis_index("s")
    wid = c * NS + s

    _zero_rows(rows_v, 8)
    pltpu.sync_copy(rows_v.at[pl.ds(0, 8)], pool_sh.at[pl.ds(s * 8, 8)])
    plsc.subcore_barrier()

    rbase = wid * RPW

    @pl.loop(0, PSTEPS)
    def _(t):
        off = pl.multiple_of(rbase + t * PCH, PCH)
        pltpu.sync_copy(h_hbm.at[pl.ds(off, PCH)], rows_v)
        pltpu.sync_copy(batch_hbm.at[pl.ds(off, PCH)], bidx_v)
        pltpu.sync_copy(rows_v, pool_sh.at[bidx_v], add=True)

    plsc.subcore_barrier()
    pltpu.sync_copy(
        pool_sh.at[pl.ds(s * 8, 8)],
        out_hbm.at[pl.ds(c * G + s * 8, 8)],
    )


BLK = 1024
NB = N_PAD // BLK


def _dense_body(h_ref, p0_ref, p1_ref, w1_ref, b1_ref, g_ref, be_ref, w2_ref,
                b2_ref, o_ref):
    z = h_ref[...] + p0_ref[...] + p1_ref[...]
    z = jnp.dot(z, w1_ref[...], preferred_element_type=jnp.float32) + b1_ref[...]
    z = z * (g_ref[...] * _BN_INV) + be_ref[...]
    z = jnp.maximum(z, 0.0)
    z = jnp.dot(z, w2_ref[...], preferred_element_type=jnp.float32) + b2_ref[...]
    o_ref[...] = jnp.maximum(z, 0.0)


def _tc_dense(h, parts, W1, b1, gamma, beta, W2, b2):
    vspec = pl.BlockSpec((1, D), lambda i: (0, 0))
    wspec = pl.BlockSpec((D, D), lambda i: (0, 0))
    return pl.pallas_call(
        _dense_body,
        grid=(NB,),
        in_specs=[
            pl.BlockSpec((BLK, D), lambda i: (i, 0)),
            pl.BlockSpec((BLK, D), lambda i: (i, 0)),
            pl.BlockSpec((BLK, D), lambda i: (i + NB, 0)),
            wspec, vspec, vspec, vspec, wspec, vspec,
        ],
        out_specs=pl.BlockSpec((BLK, D), lambda i: (i, 0)),
        out_shape=jax.ShapeDtypeStruct((N_PAD, D), jnp.float32),
    )(h, parts, parts, W1, b1.reshape(1, D), gamma.reshape(1, D),
      beta.reshape(1, D), W2, b2.reshape(1, D))


def _head_body(p_ref, w1_ref, b1_ref, w2_ref, b2_ref, o_ref):
    z = p_ref[pl.ds(0, G), :] + p_ref[pl.ds(G, G), :]
    z = jnp.dot(z, w1_ref[...], preferred_element_type=jnp.float32) + b1_ref[...]
    z = jnp.maximum(z, 0.0)
    o_ref[...] = jnp.dot(z, w2_ref[...], preferred_element_type=jnp.float32) + b2_ref[...]


def _tc_head(pparts, W1, b1, W2, b2):
    return pl.pallas_call(
        _head_body,
        out_shape=jax.ShapeDtypeStruct((G, OUT), jnp.float32),
    )(pparts, W1, b1.reshape(1, D), W2, b2.reshape(1, OUT))


def kernel(x, edge_index, batch, conv0_W1, conv0_b1, conv0_gamma, conv0_beta,
           conv0_W2, conv0_b2, conv1_W1, conv1_b1, conv1_gamma, conv1_beta,
           conv1_W2, conv1_b2, conv2_W1, conv2_b1, conv2_gamma, conv2_beta,
           conv2_W2, conv2_b2, mlp_W1, mlp_b1, mlp_W2, mlp_b2):
    src = edge_index[0].astype(jnp.int32)
    dst = edge_index[1].astype(jnp.int32)
    srcp = jnp.pad(src, (0, E_PAD - E))
    dstp = jnp.pad(dst, (0, E_PAD - E), constant_values=N_PAD)
    batchp = jnp.pad(batch.astype(jnp.int32), (0, N_PAD - N), constant_values=G)
    h = jnp.pad(x, ((0, N_PAD - N), (0, 0)))

    layers = (
        (conv0_W1, conv0_b1, conv0_gamma, conv0_beta, conv0_W2, conv0_b2),
        (conv1_W1, conv1_b1, conv1_gamma, conv1_beta, conv1_W2, conv1_b2),
        (conv2_W1, conv2_b1, conv2_gamma, conv2_beta, conv2_W2, conv2_b2),
    )
    for (W1, b1, gamma, beta, W2, b2) in layers:
        parts = _sc_segsum(h, srcp, dstp)
        h = _tc_dense(h, parts, W1, b1, gamma, beta, W2, b2)

    pparts = _sc_pool(h, batchp)
    return _tc_head(pparts, mlp_W1, mlp_b1, mlp_W2, mlp_b2)
